# baseline (speedup 1.0000x reference)
import jax
import jax.numpy as jnp
from jax.experimental import pallas as pl

SPATIAL = (128, 128, 16)
NEG_SLOPE = 0.01
EPS = 1e-5


def _bn_leaky_body(x_ref, mu_ref, inv_ref, g_ref, b_ref, o_ref):
    x = x_ref[...]
    y = (x - mu_ref[...]) * inv_ref[...] * g_ref[...] + b_ref[...]
    o_ref[...] = jnp.where(y >= 0, y, NEG_SLOPE * y)


def _bn_leaky(x, gamma, beta):
    mu = jnp.mean(x, axis=0, keepdims=True)
    var = jnp.var(x, axis=0, keepdims=True)
    inv = 1.0 / jnp.sqrt(var + EPS)
    n = x.shape[0]
    blk = 8000
    return pl.pallas_call(
        _bn_leaky_body,
        grid=(n // blk,),
        in_specs=[
            pl.BlockSpec((blk, x.shape[1]), lambda i: (i, 0)),
            pl.BlockSpec((1, x.shape[1]), lambda i: (0, 0)),
            pl.BlockSpec((1, x.shape[1]), lambda i: (0, 0)),
            pl.BlockSpec((1, x.shape[1]), lambda i: (0, 0)),
            pl.BlockSpec((1, x.shape[1]), lambda i: (0, 0)),
        ],
        out_specs=pl.BlockSpec((blk, x.shape[1]), lambda i: (i, 0)),
        out_shape=jax.ShapeDtypeStruct(x.shape, x.dtype),
    )(x, mu, inv, gamma[None, :], beta[None, :])


def _build_neighbors(coors):
    sx, sy, sz = SPATIAL
    n = coors.shape[0]
    x = coors[:, 1].astype(jnp.int32)
    y = coors[:, 2].astype(jnp.int32)
    z = coors[:, 3].astype(jnp.int32)
    lin = x * (sy * sz) + y * sz + z
    grid = jnp.full((sx * sy * sz,), -1, dtype=jnp.int32).at[lin].set(
        jnp.arange(n, dtype=jnp.int32))
    nbrs = []
    for dx in (-1, 0, 1):
        for dy in (-1, 0, 1):
            for dz in (-1, 0, 1):
                nx, ny, nz = x + dx, y + dy, z + dz
                valid = (nx >= 0) & (nx < sx) & (ny >= 0) & (ny < sy) & (nz >= 0) & (nz < sz)
                nlin = jnp.clip(nx * (sy * sz) + ny * sz + nz, 0, sx * sy * sz - 1)
                nbrs.append(jnp.where(valid, grid[nlin], -1))
    return jnp.stack(nbrs)


def _subm_conv(feat, nbrs, W):
    n = feat.shape[0]
    out = jnp.zeros((n, W.shape[-1]), feat.dtype)
    for k in range(27):
        nidx = nbrs[k]
        safe = jnp.clip(nidx, 0, n - 1)
        g = jnp.where((nidx >= 0)[:, None], feat[safe], 0.0)
        out = out + g @ W[k]
    return out


def kernel(features, coors_inv_last, coors_inv, coors, W1, g1, b1, W2, g2, b2):
    n_v = coors.shape[0]
    gathered = features[coors_inv_last]
    sums = jax.ops.segment_sum(gathered, coors_inv, num_segments=n_v)
    cnt = jax.ops.segment_sum(jnp.ones((gathered.shape[0], 1), jnp.float32),
                              coors_inv, num_segments=n_v)
    v_fea = sums / jnp.maximum(cnt, 1.0)
    nbrs = _build_neighbors(coors)
    out = _subm_conv(v_fea, nbrs, W1)
    out = _bn_leaky(out, g1, b1)
    out = _subm_conv(out, nbrs, W2)
    mu = jnp.mean(out, axis=0)
    var = jnp.var(out, axis=0)
    out = (out - mu) / jnp.sqrt(var + EPS) * g2 + b2
    out = out + v_fea
    return jnp.where(out >= 0, out, NEG_SLOPE * out)


# speedup vs baseline: 2.7321x; 2.7321x over previous
import functools

import jax
import jax.numpy as jnp
from jax import lax
from jax.experimental import pallas as pl
from jax.experimental.pallas import tpu as pltpu

SX, SY, SZ = 128, 128, 16
X2, Y2 = SX + 2, SY + 2
NROWS = X2 * Y2 * SZ
R = 2704
NBLK = NROWS // R
C = 64
NEG_SLOPE = 0.01
EPS = 1e-5

_OFFSETS = [(dx, dy, dz)
            for dx in (-1, 0, 1) for dy in (-1, 0, 1) for dz in (-1, 0, 1)]


def _zmask(jz, dz, a):
    if dz == 1:
        return jnp.where(jz != SZ - 1, a, 0.0)
    if dz == -1:
        return jnp.where(jz != 0, a, 0.0)
    return a


def _accumulate(w_ref, scratch):
    jz = lax.broadcasted_iota(jnp.int32, (R, 1), 0) % SZ
    acc = jnp.zeros((R, C), jnp.float32)
    for k, (dx, dy, dz) in enumerate(_OFFSETS):
        d = (dx * Y2 + dy) * SZ + dz
        a = scratch[pl.ds(R + d, R), :]
        a = _zmask(jz, dz, a)
        acc = acc + jnp.dot(a, w_ref[pl.ds(k * C, C), :],
                            preferred_element_type=jnp.float32)
    return acc


def _stats(o_ref, s1_ref, s2_ref, mult_ref, acc):
    o_ref[...] = acc
    m = mult_ref[...]
    s1_ref[...] = jnp.sum(acc * m, axis=0, keepdims=True)[None]
    s2_ref[...] = jnp.sum(acc * acc * m, axis=0, keepdims=True)[None]


def _conv1_body(w_ref, mult_ref, prev_ref, cur_ref, nxt_ref,
                o_ref, s1_ref, s2_ref, scratch):
    scratch[pl.ds(0, R), :] = prev_ref[...]
    scratch[pl.ds(R, R), :] = cur_ref[...]
    scratch[pl.ds(2 * R, R), :] = nxt_ref[...]
    acc = _accumulate(w_ref, scratch)
    _stats(o_ref, s1_ref, s2_ref, mult_ref, acc)


def _conv2_body(w_ref, mult_ref, mu_ref, inv_ref, g_ref, b_ref,
                prev_ref, cur_ref, nxt_ref,
                oprev_ref, ocur_ref, onxt_ref,
                o_ref, s1_ref, s2_ref, scratch):
    def prep(x, occ):
        y = (x - mu_ref[...]) * inv_ref[...] * g_ref[...] + b_ref[...]
        y = jnp.where(y >= 0, y, NEG_SLOPE * y)
        return y * occ

    scratch[pl.ds(0, R), :] = prep(prev_ref[...], oprev_ref[...])
    scratch[pl.ds(R, R), :] = prep(cur_ref[...], ocur_ref[...])
    scratch[pl.ds(2 * R, R), :] = prep(nxt_ref[...], onxt_ref[...])
    acc = _accumulate(w_ref, scratch)
    _stats(o_ref, s1_ref, s2_ref, mult_ref, acc)


_ROWSPEC = dict(
    prev=pl.BlockSpec((R, C), lambda i: (jnp.maximum(i - 1, 0), 0)),
    cur=pl.BlockSpec((R, C), lambda i: (i, 0)),
    nxt=pl.BlockSpec((R, C), lambda i: (jnp.minimum(i + 1, NBLK - 1), 0)),
)


def _conv1(grid_feat, mult, w):
    return pl.pallas_call(
        _conv1_body,
        grid=(NBLK,),
        in_specs=[
            pl.BlockSpec((27 * C, C), lambda i: (0, 0)),
            pl.BlockSpec((R, 1), lambda i: (i, 0)),
            _ROWSPEC["prev"], _ROWSPEC["cur"], _ROWSPEC["nxt"],
        ],
        out_specs=[
            pl.BlockSpec((R, C), lambda i: (i, 0)),
            pl.BlockSpec((1, 1, C), lambda i: (i, 0, 0)),
            pl.BlockSpec((1, 1, C), lambda i: (i, 0, 0)),
        ],
        out_shape=[
            jax.ShapeDtypeStruct((NROWS, C), jnp.float32),
            jax.ShapeDtypeStruct((NBLK, 1, C), jnp.float32),
            jax.ShapeDtypeStruct((NBLK, 1, C), jnp.float32),
        ],
        scratch_shapes=[pltpu.VMEM((3 * R, C), jnp.float32)],
        compiler_params=pltpu.CompilerParams(
            dimension_semantics=("arbitrary",)),
    )(w, mult, grid_feat, grid_feat, grid_feat)


def _conv2(grid_feat, occ, mult, w, mu, inv, g, b):
    return pl.pallas_call(
        _conv2_body,
        grid=(NBLK,),
        in_specs=[
            pl.BlockSpec((27 * C, C), lambda i: (0, 0)),
            pl.BlockSpec((R, 1), lambda i: (i, 0)),
            pl.BlockSpec((1, C), lambda i: (0, 0)),
            pl.BlockSpec((1, C), lambda i: (0, 0)),
            pl.BlockSpec((1, C), lambda i: (0, 0)),
            pl.BlockSpec((1, C), lambda i: (0, 0)),
            _ROWSPEC["prev"], _ROWSPEC["cur"], _ROWSPEC["nxt"],
            pl.BlockSpec((R, 1), lambda i: (jnp.maximum(i - 1, 0), 0)),
            pl.BlockSpec((R, 1), lambda i: (i, 0)),
            pl.BlockSpec((R, 1), lambda i: (jnp.minimum(i + 1, NBLK - 1), 0)),
        ],
        out_specs=[
            pl.BlockSpec((R, C), lambda i: (i, 0)),
            pl.BlockSpec((1, 1, C), lambda i: (i, 0, 0)),
            pl.BlockSpec((1, 1, C), lambda i: (i, 0, 0)),
        ],
        out_shape=[
            jax.ShapeDtypeStruct((NROWS, C), jnp.float32),
            jax.ShapeDtypeStruct((NBLK, 1, C), jnp.float32),
            jax.ShapeDtypeStruct((NBLK, 1, C), jnp.float32),
        ],
        scratch_shapes=[pltpu.VMEM((3 * R, C), jnp.float32)],
        compiler_params=pltpu.CompilerParams(
            dimension_semantics=("arbitrary",)),
    )(w, mult, mu, inv, g, b, grid_feat, grid_feat, grid_feat, occ, occ, occ)


def _final_body(t_ref, v_ref, mu_ref, inv_ref, g_ref, b_ref, o_ref):
    y = (t_ref[...] - mu_ref[...]) * inv_ref[...] * g_ref[...] + b_ref[...]
    y = y + v_ref[...]
    o_ref[...] = jnp.where(y >= 0, y, NEG_SLOPE * y)


def _final(t, v_fea, mu, inv, g, b):
    n = t.shape[0]
    blk = 8000
    vspec = pl.BlockSpec((1, C), lambda i: (0, 0))
    return pl.pallas_call(
        _final_body,
        grid=(n // blk,),
        in_specs=[pl.BlockSpec((blk, C), lambda i: (i, 0)),
                  pl.BlockSpec((blk, C), lambda i: (i, 0)),
                  vspec, vspec, vspec, vspec],
        out_specs=pl.BlockSpec((blk, C), lambda i: (i, 0)),
        out_shape=jax.ShapeDtypeStruct((n, C), jnp.float32),
    )(t, v_fea, mu, inv, g, b)


def _finalize_stats(s1, s2, n_v):
    mu = jnp.sum(s1[:, 0, :], axis=0, keepdims=True) / n_v
    ex2 = jnp.sum(s2[:, 0, :], axis=0, keepdims=True) / n_v
    var = ex2 - mu * mu
    inv = 1.0 / jnp.sqrt(var + EPS)
    return mu, inv


def kernel(features, coors_inv_last, coors_inv, coors, W1, g1, b1, W2, g2, b2):
    n_v = coors.shape[0]
    n_pts = coors_inv.shape[0]

    gathered = features[coors_inv_last]
    sums = jax.ops.segment_sum(gathered, coors_inv, num_segments=n_v)
    cnt = jax.ops.segment_sum(jnp.ones((n_pts, 1), jnp.float32),
                              coors_inv, num_segments=n_v)
    v_fea = sums / jnp.maximum(cnt, 1.0)

    cx = coors[:, 1].astype(jnp.int32)
    cy = coors[:, 2].astype(jnp.int32)
    cz = coors[:, 3].astype(jnp.int32)
    lin = cx * (SY * SZ) + cy * SZ + cz
    grid_idx = jnp.full((SX * SY * SZ,), -1, jnp.int32).at[lin].set(
        jnp.arange(n_v, dtype=jnp.int32))
    cnt_cell = jnp.zeros((SX * SY * SZ,), jnp.float32).at[lin].add(1.0)

    idx_pad = jnp.pad(grid_idx.reshape(SX, SY, SZ),
                      ((1, 1), (1, 1), (0, 0)), constant_values=-1).reshape(-1)
    mult = jnp.pad(cnt_cell.reshape(SX, SY, SZ),
                   ((1, 1), (1, 1), (0, 0))).reshape(-1, 1)
    occ = (idx_pad >= 0).astype(jnp.float32)[:, None]

    safe = jnp.where(idx_pad < 0, n_v, idx_pad)
    v_ext = jnp.concatenate([v_fea, jnp.zeros((1, C), jnp.float32)], axis=0)
    grid_feat = v_ext[safe]

    w1 = W1.reshape(27 * C, C)
    w2 = W2.reshape(27 * C, C)

    o1, s1a, s1b = _conv1(grid_feat, mult, w1)
    mu1, inv1 = _finalize_stats(s1a, s1b, float(n_v))

    o2, s2a, s2b = _conv2(o1, occ, mult, w2, mu1, inv1,
                          g1[None, :], b1[None, :])
    mu2, inv2 = _finalize_stats(s2a, s2b, float(n_v))

    lin_pad = ((cx + 1) * Y2 + (cy + 1)) * SZ + cz
    t = o2[lin_pad]

    return _final(t, v_fea, mu2, inv2, g2[None, :], b2[None, :])


# speedup vs baseline: 2.8639x; 1.0483x over previous
import functools

import jax
import jax.numpy as jnp
from jax import lax
from jax.experimental import pallas as pl
from jax.experimental.pallas import tpu as pltpu

SX, SY, SZ = 128, 128, 16
X2, Y2 = SX + 2, SY + 2
NROWS = X2 * Y2 * SZ
R = 2704
NBLK = NROWS // R
C = 64
NEG_SLOPE = 0.01
EPS = 1e-5

_OFFSETS = [(dx, dy, dz)
            for dx in (-1, 0, 1) for dy in (-1, 0, 1) for dz in (-1, 0, 1)]


def _zmask(jz, dz, a):
    if dz == 1:
        return jnp.where(jz != SZ - 1, a, 0.0)
    if dz == -1:
        return jnp.where(jz != 0, a, 0.0)
    return a


def _accumulate(w_ref, scratch):
    jz = lax.broadcasted_iota(jnp.int32, (R, 1), 0) % SZ
    slices = []
    for dx, dy, dz in _OFFSETS:
        d = (dx * Y2 + dy) * SZ + dz
        a = scratch[pl.ds(R + d, R), :]
        slices.append(_zmask(jz, dz, a))
    a27 = jnp.concatenate(slices, axis=1)
    return jnp.dot(a27, w_ref[...], preferred_element_type=jnp.float32)


def _stats(o_ref, s1_ref, s2_ref, mult_ref, acc):
    o_ref[...] = acc
    m = mult_ref[...]
    s1_ref[...] = jnp.sum(acc * m, axis=0, keepdims=True)[None]
    s2_ref[...] = jnp.sum(acc * acc * m, axis=0, keepdims=True)[None]


def _conv1_body(w_ref, mult_ref, prev_ref, cur_ref, nxt_ref,
                o_ref, s1_ref, s2_ref, scratch):
    scratch[pl.ds(0, R), :] = prev_ref[...]
    scratch[pl.ds(R, R), :] = cur_ref[...]
    scratch[pl.ds(2 * R, R), :] = nxt_ref[...]
    acc = _accumulate(w_ref, scratch)
    _stats(o_ref, s1_ref, s2_ref, mult_ref, acc)


def _conv2_body(w_ref, mult_ref, mu_ref, inv_ref, g_ref, b_ref,
                prev_ref, cur_ref, nxt_ref,
                oprev_ref, ocur_ref, onxt_ref,
                o_ref, s1_ref, s2_ref, scratch):
    def prep(x, occ):
        y = (x - mu_ref[...]) * inv_ref[...] * g_ref[...] + b_ref[...]
        y = jnp.where(y >= 0, y, NEG_SLOPE * y)
        return y * occ

    scratch[pl.ds(0, R), :] = prep(prev_ref[...], oprev_ref[...])
    scratch[pl.ds(R, R), :] = prep(cur_ref[...], ocur_ref[...])
    scratch[pl.ds(2 * R, R), :] = prep(nxt_ref[...], onxt_ref[...])
    acc = _accumulate(w_ref, scratch)
    _stats(o_ref, s1_ref, s2_ref, mult_ref, acc)


_ROWSPEC = dict(
    prev=pl.BlockSpec((R, C), lambda i: (jnp.maximum(i - 1, 0), 0)),
    cur=pl.BlockSpec((R, C), lambda i: (i, 0)),
    nxt=pl.BlockSpec((R, C), lambda i: (jnp.minimum(i + 1, NBLK - 1), 0)),
)


def _conv1(grid_feat, mult, w):
    return pl.pallas_call(
        _conv1_body,
        grid=(NBLK,),
        in_specs=[
            pl.BlockSpec((27 * C, C), lambda i: (0, 0)),
            pl.BlockSpec((R, 1), lambda i: (i, 0)),
            _ROWSPEC["prev"], _ROWSPEC["cur"], _ROWSPEC["nxt"],
        ],
        out_specs=[
            pl.BlockSpec((R, C), lambda i: (i, 0)),
            pl.BlockSpec((1, 1, C), lambda i: (i, 0, 0)),
            pl.BlockSpec((1, 1, C), lambda i: (i, 0, 0)),
        ],
        out_shape=[
            jax.ShapeDtypeStruct((NROWS, C), jnp.float32),
            jax.ShapeDtypeStruct((NBLK, 1, C), jnp.float32),
            jax.ShapeDtypeStruct((NBLK, 1, C), jnp.float32),
        ],
        scratch_shapes=[pltpu.VMEM((3 * R, C), jnp.float32)],
        compiler_params=pltpu.CompilerParams(
            dimension_semantics=("arbitrary",)),
    )(w, mult, grid_feat, grid_feat, grid_feat)


def _conv2(grid_feat, occ, mult, w, mu, inv, g, b):
    return pl.pallas_call(
        _conv2_body,
        grid=(NBLK,),
        in_specs=[
            pl.BlockSpec((27 * C, C), lambda i: (0, 0)),
            pl.BlockSpec((R, 1), lambda i: (i, 0)),
            pl.BlockSpec((1, C), lambda i: (0, 0)),
            pl.BlockSpec((1, C), lambda i: (0, 0)),
            pl.BlockSpec((1, C), lambda i: (0, 0)),
            pl.BlockSpec((1, C), lambda i: (0, 0)),
            _ROWSPEC["prev"], _ROWSPEC["cur"], _ROWSPEC["nxt"],
            pl.BlockSpec((R, 1), lambda i: (jnp.maximum(i - 1, 0), 0)),
            pl.BlockSpec((R, 1), lambda i: (i, 0)),
            pl.BlockSpec((R, 1), lambda i: (jnp.minimum(i + 1, NBLK - 1), 0)),
        ],
        out_specs=[
            pl.BlockSpec((R, C), lambda i: (i, 0)),
            pl.BlockSpec((1, 1, C), lambda i: (i, 0, 0)),
            pl.BlockSpec((1, 1, C), lambda i: (i, 0, 0)),
        ],
        out_shape=[
            jax.ShapeDtypeStruct((NROWS, C), jnp.float32),
            jax.ShapeDtypeStruct((NBLK, 1, C), jnp.float32),
            jax.ShapeDtypeStruct((NBLK, 1, C), jnp.float32),
        ],
        scratch_shapes=[pltpu.VMEM((3 * R, C), jnp.float32)],
        compiler_params=pltpu.CompilerParams(
            dimension_semantics=("arbitrary",)),
    )(w, mult, mu, inv, g, b, grid_feat, grid_feat, grid_feat, occ, occ, occ)


def _final_body(t_ref, v_ref, mu_ref, inv_ref, g_ref, b_ref, o_ref):
    y = (t_ref[...] - mu_ref[...]) * inv_ref[...] * g_ref[...] + b_ref[...]
    y = y + v_ref[...]
    o_ref[...] = jnp.where(y >= 0, y, NEG_SLOPE * y)


def _final(t, v_fea, mu, inv, g, b):
    n = t.shape[0]
    blk = 8000
    vspec = pl.BlockSpec((1, C), lambda i: (0, 0))
    return pl.pallas_call(
        _final_body,
        grid=(n // blk,),
        in_specs=[pl.BlockSpec((blk, C), lambda i: (i, 0)),
                  pl.BlockSpec((blk, C), lambda i: (i, 0)),
                  vspec, vspec, vspec, vspec],
        out_specs=pl.BlockSpec((blk, C), lambda i: (i, 0)),
        out_shape=jax.ShapeDtypeStruct((n, C), jnp.float32),
    )(t, v_fea, mu, inv, g, b)


def _finalize_stats(s1, s2, n_v):
    mu = jnp.sum(s1[:, 0, :], axis=0, keepdims=True) / n_v
    ex2 = jnp.sum(s2[:, 0, :], axis=0, keepdims=True) / n_v
    var = ex2 - mu * mu
    inv = 1.0 / jnp.sqrt(var + EPS)
    return mu, inv


def kernel(features, coors_inv_last, coors_inv, coors, W1, g1, b1, W2, g2, b2):
    n_v = coors.shape[0]
    n_pts = coors_inv.shape[0]

    gathered = features[coors_inv_last]
    sums = jax.ops.segment_sum(gathered, coors_inv, num_segments=n_v)
    cnt = jax.ops.segment_sum(jnp.ones((n_pts, 1), jnp.float32),
                              coors_inv, num_segments=n_v)
    v_fea = sums / jnp.maximum(cnt, 1.0)

    cx = coors[:, 1].astype(jnp.int32)
    cy = coors[:, 2].astype(jnp.int32)
    cz = coors[:, 3].astype(jnp.int32)
    lin = cx * (SY * SZ) + cy * SZ + cz
    grid_idx = jnp.full((SX * SY * SZ,), -1, jnp.int32).at[lin].set(
        jnp.arange(n_v, dtype=jnp.int32))
    cnt_cell = jnp.zeros((SX * SY * SZ,), jnp.float32).at[lin].add(1.0)

    idx_pad = jnp.pad(grid_idx.reshape(SX, SY, SZ),
                      ((1, 1), (1, 1), (0, 0)), constant_values=-1).reshape(-1)
    mult = jnp.pad(cnt_cell.reshape(SX, SY, SZ),
                   ((1, 1), (1, 1), (0, 0))).reshape(-1, 1)
    occ = (idx_pad >= 0).astype(jnp.float32)[:, None]

    safe = jnp.where(idx_pad < 0, n_v, idx_pad)
    v_ext = jnp.concatenate([v_fea, jnp.zeros((1, C), jnp.float32)], axis=0)
    grid_feat = v_ext[safe]

    w1 = W1.reshape(27 * C, C)
    w2 = W2.reshape(27 * C, C)

    o1, s1a, s1b = _conv1(grid_feat, mult, w1)
    mu1, inv1 = _finalize_stats(s1a, s1b, float(n_v))

    o2, s2a, s2b = _conv2(o1, occ, mult, w2, mu1, inv1,
                          g1[None, :], b1[None, :])
    mu2, inv2 = _finalize_stats(s2a, s2b, float(n_v))

    lin_pad = ((cx + 1) * Y2 + (cy + 1)) * SZ + cz
    t = o2[lin_pad]

    return _final(t, v_fea, mu2, inv2, g2[None, :], b2[None, :])
